# Initial kernel scaffold; baseline (speedup 1.0000x reference)
import jax
import jax.numpy as jnp
from jax.experimental import pallas as pl


def kernel(v0, v1, v2, v3, edge_index0, edge_index1, edge_index2, W_lin, bias, W1, b1, W2, b2):
    raise NotImplementedError("write your pallas kernel here")


# speedup vs baseline: 6.3049x; 6.3049x over previous
import functools

import jax
import jax.numpy as jnp
from jax import lax
from jax.experimental import pallas as pl
from jax.experimental.pallas import tpu as pltpu
from jax.experimental.pallas import tpu_sc as plsc

N = 10000
E = 320000
F = 128


def _lin_body(v0_ref, wl_ref, x_ref):
    x_ref[...] = jnp.dot(v0_ref[...], wl_ref[...],
                         preferred_element_type=jnp.float32)


def _lin(v0, W_lin):
    return pl.pallas_call(
        _lin_body,
        out_shape=jax.ShapeDtypeStruct((N, F), jnp.float32),
        grid=(10,),
        in_specs=[
            pl.BlockSpec((N // 10, F), lambda i: (i, 0)),
            pl.BlockSpec((F, F), lambda i: (0, 0)),
        ],
        out_specs=pl.BlockSpec((N // 10, F), lambda i: (i, 0)),
    )(v0, W_lin)


_MLP_BM = 2000


def _mlp_body(v1_ref, w1_ref, b1_ref, w2_ref, b2_ref, w_ref):
    h = jnp.dot(v1_ref[...], w1_ref[...], preferred_element_type=jnp.float32)
    h = jnp.maximum(h + b1_ref[...], 0.0)
    w_ref[...] = jnp.dot(h, w2_ref[...],
                         preferred_element_type=jnp.float32) + b2_ref[...]


def _mlp(v1, W1, b1, W2, b2):
    ef = v1.shape[1]
    return pl.pallas_call(
        _mlp_body,
        out_shape=jax.ShapeDtypeStruct((E, F), jnp.float32),
        grid=(E // _MLP_BM,),
        in_specs=[
            pl.BlockSpec((_MLP_BM, ef), lambda i: (i, 0)),
            pl.BlockSpec((ef, F), lambda i: (0, 0)),
            pl.BlockSpec((1, F), lambda i: (0, 0)),
            pl.BlockSpec((F, F), lambda i: (0, 0)),
            pl.BlockSpec((1, F), lambda i: (0, 0)),
        ],
        out_specs=pl.BlockSpec((_MLP_BM, F), lambda i: (i, 0)),
    )(v1, W1, b1.reshape(1, F), W2, b2.reshape(1, F))


_NW = 32
_C = 80
_EPT = E // _NW
_NCHUNK = _EPT // _C
_RSTRIDE = 624
_RSIZE = 640


def _sc_agg(ei, ej, x, w):
    mesh = plsc.VectorSubcoreMesh(core_axis_name="c", subcore_axis_name="s")

    @functools.partial(
        pl.kernel,
        mesh=mesh,
        out_type=jax.ShapeDtypeStruct((2, N, F), jnp.float32),
        scratch_types=[
            pltpu.VMEM((_C,), jnp.int32),
            pltpu.VMEM((_C,), jnp.int32),
            pltpu.VMEM((_C, F), jnp.float32),
            pltpu.VMEM((_C, F), jnp.float32),
            pltpu.VMEM((_C, F), jnp.float32),
            pltpu.VMEM_SHARED((N, F), jnp.float32),
            pltpu.SemaphoreType.DMA,
            pltpu.SemaphoreType.DMA,
        ],
    )
    def k(ei_hbm, ej_hbm, x_hbm, w_hbm, out_hbm,
          ii_v, jj_v, xi_v, xj_v, w_v, acc_sh, sem1, sem2):
        cid = lax.axis_index("c")
        sid = lax.axis_index("s")
        wid = sid * 2 + cid

        def zbody(r, carry):
            for kk in range(8):
                xi_v[r, pl.ds(kk * 16, 16)] = jnp.zeros((16,), jnp.float32)
            return carry

        lax.fori_loop(0, _C, zbody, 0)
        zbase = sid * _RSTRIDE
        for t in range(_RSIZE // _C):
            pltpu.sync_copy(xi_v, acc_sh.at[pl.ds(zbase + t * _C, _C)])
        plsc.subcore_barrier()

        ebase = wid * _EPT

        def chunk(c, carry):
            b = ebase + c * _C
            pltpu.sync_copy(ei_hbm.at[pl.ds(b, _C)], ii_v)
            pltpu.sync_copy(ej_hbm.at[pl.ds(b, _C)], jj_v)
            cp1 = pltpu.async_copy(x_hbm.at[jj_v], xj_v, sem1)
            cp2 = pltpu.async_copy(x_hbm.at[ii_v], xi_v, sem2)
            pltpu.sync_copy(w_hbm.at[pl.ds(b, _C)], w_v)
            cp1.wait()
            cp2.wait()

            def mbody(r, carry2):
                for kk in range(8):
                    sl = pl.ds(kk * 16, 16)
                    wv = w_v[r, sl]
                    xj_v[r, sl] = xj_v[r, sl] * wv
                    xi_v[r, sl] = xi_v[r, sl] * wv
                return carry2

            lax.fori_loop(0, _C, mbody, 0)
            pltpu.sync_copy(xj_v, acc_sh.at[ii_v], add=True)
            pltpu.sync_copy(xi_v, acc_sh.at[jj_v], add=True)
            return carry

        lax.fori_loop(0, _NCHUNK, chunk, 0)
        plsc.subcore_barrier()

        obase = sid * _RSTRIDE
        pltpu.sync_copy(acc_sh.at[pl.ds(obase, _RSIZE)],
                        out_hbm.at[cid, pl.ds(obase, _RSIZE)])

    return k(ei, ej, x, w)


def _comb_body(p_ref, b_ref, o_ref):
    o_ref[...] = jnp.maximum(p_ref[0] + p_ref[1] + b_ref[...], 0.0)


def _combine(partials, bias):
    return pl.pallas_call(
        _comb_body,
        out_shape=jax.ShapeDtypeStruct((N, F), jnp.float32),
        grid=(10,),
        in_specs=[
            pl.BlockSpec((2, N // 10, F), lambda i: (0, i, 0)),
            pl.BlockSpec((1, F), lambda i: (0, 0)),
        ],
        out_specs=pl.BlockSpec((N // 10, F), lambda i: (i, 0)),
    )(partials, bias.reshape(1, F))


def kernel(v0, v1, v2, v3, edge_index0, edge_index1, edge_index2,
           W_lin, bias, W1, b1, W2, b2):
    x = _lin(v0, W_lin)
    w = _mlp(v1, W1, b1, W2, b2)
    ei = edge_index0[0].astype(jnp.int32)
    ej = edge_index0[1].astype(jnp.int32)
    partials = _sc_agg(ei, ej, x, w)
    return _combine(partials, bias)


# speedup vs baseline: 6.7674x; 1.0734x over previous
import functools

import jax
import jax.numpy as jnp
from jax import lax
from jax.experimental import pallas as pl
from jax.experimental.pallas import tpu as pltpu
from jax.experimental.pallas import tpu_sc as plsc

N = 10000
E = 320000
F = 128


def _lin_body(v0_ref, wl_ref, x_ref):
    x_ref[...] = jnp.dot(v0_ref[...], wl_ref[...],
                         preferred_element_type=jnp.float32)


def _lin(v0, W_lin):
    return pl.pallas_call(
        _lin_body,
        out_shape=jax.ShapeDtypeStruct((N, F), jnp.float32),
        grid=(10,),
        in_specs=[
            pl.BlockSpec((N // 10, F), lambda i: (i, 0)),
            pl.BlockSpec((F, F), lambda i: (0, 0)),
        ],
        out_specs=pl.BlockSpec((N // 10, F), lambda i: (i, 0)),
    )(v0, W_lin)


_MLP_BM = 2000


def _mlp_body(v1_ref, w1_ref, b1_ref, w2_ref, b2_ref, w_ref):
    h = jnp.dot(v1_ref[...], w1_ref[...], preferred_element_type=jnp.float32)
    h = jnp.maximum(h + b1_ref[...], 0.0)
    w_ref[...] = jnp.dot(h, w2_ref[...],
                         preferred_element_type=jnp.float32) + b2_ref[...]


def _mlp(v1, W1, b1, W2, b2):
    ef = v1.shape[1]
    return pl.pallas_call(
        _mlp_body,
        out_shape=jax.ShapeDtypeStruct((E, F), jnp.float32),
        grid=(E // _MLP_BM,),
        in_specs=[
            pl.BlockSpec((_MLP_BM, ef), lambda i: (i, 0)),
            pl.BlockSpec((ef, F), lambda i: (0, 0)),
            pl.BlockSpec((1, F), lambda i: (0, 0)),
            pl.BlockSpec((F, F), lambda i: (0, 0)),
            pl.BlockSpec((1, F), lambda i: (0, 0)),
        ],
        out_specs=pl.BlockSpec((_MLP_BM, F), lambda i: (i, 0)),
    )(v1, W1, b1.reshape(1, F), W2, b2.reshape(1, F))


_NW = 32
_C = 40
_EPT = E // _NW
_NCHUNK = _EPT // _C
_RSTRIDE = 624
_RSIZE = 640


def _sc_agg(ei, ej, x, w):
    mesh = plsc.VectorSubcoreMesh(core_axis_name="c", subcore_axis_name="s")

    @functools.partial(
        pl.kernel,
        mesh=mesh,
        out_type=jax.ShapeDtypeStruct((2, N, F), jnp.float32),
        scratch_types=(
            [pltpu.VMEM((_C,), jnp.int32)] * 6
            + [pltpu.VMEM((_C, F), jnp.float32)] * 9
            + [pltpu.VMEM_SHARED((N, F), jnp.float32)]
            + [pltpu.SemaphoreType.DMA] * 12
        ),
    )
    def k(ei_hbm, ej_hbm, x_hbm, w_hbm, out_hbm,
          ii0, ii1, ii2, jj0, jj1, jj2,
          w0, w1, w2, xi0, xi1, xi2, xj0, xj1, xj2,
          acc_sh,
          si0, si1, si2, sw0, sw1, sw2,
          sg0, sg1, sg2, ss0, ss1, ss2):
        IB = [ii0, ii1, ii2]
        JB = [jj0, jj1, jj2]
        WB = [w0, w1, w2]
        XIB = [xi0, xi1, xi2]
        XJB = [xj0, xj1, xj2]
        SIN = [si0, si1, si2]
        SW = [sw0, sw1, sw2]
        SG = [sg0, sg1, sg2]
        SS = [ss0, ss1, ss2]

        cid = lax.axis_index("c")
        sid = lax.axis_index("s")
        wid = sid * 2 + cid

        def zbody(r, carry):
            for kk in range(8):
                xi0[r, pl.ds(kk * 16, 16)] = jnp.zeros((16,), jnp.float32)
            return carry

        lax.fori_loop(0, _C, zbody, 0)
        zbase = sid * _RSTRIDE
        for t in range(_RSIZE // _C):
            pltpu.sync_copy(xi0, acc_sh.at[pl.ds(zbase + t * _C, _C)])
        plsc.subcore_barrier()

        ebase = wid * _EPT

        def issue_idx(c, p):
            b = ebase + c * _C
            pltpu.async_copy(ei_hbm.at[pl.ds(b, _C)], IB[p], SIN[p])
            pltpu.async_copy(ej_hbm.at[pl.ds(b, _C)], JB[p], SIN[p])

        def wait_idx(p):
            pltpu.make_async_copy(ei_hbm.at[pl.ds(0, _C)], IB[p], SIN[p]).wait()
            pltpu.make_async_copy(ej_hbm.at[pl.ds(0, _C)], JB[p], SIN[p]).wait()

        def issue_w(c, p):
            b = ebase + c * _C
            pltpu.async_copy(w_hbm.at[pl.ds(b, _C)], WB[p], SW[p])

        def wait_w(p):
            pltpu.make_async_copy(w_hbm.at[pl.ds(0, _C)], WB[p], SW[p]).wait()

        def issue_gather(p):
            pltpu.async_copy(x_hbm.at[JB[p]], XJB[p], SG[p])
            pltpu.async_copy(x_hbm.at[IB[p]], XIB[p], SG[p])

        def wait_gather(p):
            pltpu.make_async_copy(x_hbm.at[JB[p]], XJB[p], SG[p]).wait()
            pltpu.make_async_copy(x_hbm.at[IB[p]], XIB[p], SG[p]).wait()

        def issue_scatter(p):
            pltpu.async_copy(XJB[p], acc_sh.at[IB[p]], SS[p], add=True)
            pltpu.async_copy(XIB[p], acc_sh.at[JB[p]], SS[p], add=True)

        def wait_scatter(p):
            pltpu.make_async_copy(XJB[p], acc_sh.at[IB[p]], SS[p]).wait()
            pltpu.make_async_copy(XIB[p], acc_sh.at[JB[p]], SS[p]).wait()

        def compute(p):
            wv, xiv, xjv = WB[p], XIB[p], XJB[p]

            def mbody(r, carry):
                for rr in range(2):
                    row = 2 * r + rr
                    for kk in range(8):
                        sl = pl.ds(kk * 16, 16)
                        ww = wv[row, sl]
                        xjv[row, sl] = xjv[row, sl] * ww
                        xiv[row, sl] = xiv[row, sl] * ww
                return carry

            lax.fori_loop(0, _C // 2, mbody, 0)

        def body(c, p, first=False, prefetch=True):
            if not first:
                wait_scatter((p + 1) % 3)
            wait_gather(p)
            wait_w(p)
            compute(p)
            issue_scatter(p)
            if prefetch:
                pn = (p + 1) % 3
                issue_idx(c + 1, pn)
                issue_w(c + 1, pn)
                wait_idx(pn)
                issue_gather(pn)

        issue_idx(0, 0)
        issue_w(0, 0)
        wait_idx(0)
        issue_gather(0)

        body(0, 0, first=True)
        body(1, 1, first=True)

        ntrip = (_NCHUNK - 2) // 3
        tail_start = 2 + 3 * ntrip

        def triple(t, carry):
            for u in range(3):
                body(2 + 3 * t + u, (2 + u) % 3)
            return carry

        lax.fori_loop(0, ntrip, triple, 0)

        for c in range(tail_start, _NCHUNK):
            body(c, c % 3, prefetch=(c < _NCHUNK - 1))
        wait_scatter((_NCHUNK - 2) % 3)
        wait_scatter((_NCHUNK - 1) % 3)

        plsc.subcore_barrier()

        obase = sid * _RSTRIDE
        pltpu.sync_copy(acc_sh.at[pl.ds(obase, _RSIZE)],
                        out_hbm.at[cid, pl.ds(obase, _RSIZE)])

    return k(ei, ej, x, w)


def _comb_body(p_ref, b_ref, o_ref):
    o_ref[...] = jnp.maximum(p_ref[0] + p_ref[1] + b_ref[...], 0.0)


def _combine(partials, bias):
    return pl.pallas_call(
        _comb_body,
        out_shape=jax.ShapeDtypeStruct((N, F), jnp.float32),
        grid=(10,),
        in_specs=[
            pl.BlockSpec((2, N // 10, F), lambda i: (0, i, 0)),
            pl.BlockSpec((1, F), lambda i: (0, 0)),
        ],
        out_specs=pl.BlockSpec((N // 10, F), lambda i: (i, 0)),
    )(partials, bias.reshape(1, F))


def kernel(v0, v1, v2, v3, edge_index0, edge_index1, edge_index2,
           W_lin, bias, W1, b1, W2, b2):
    x = _lin(v0, W_lin)
    w = _mlp(v1, W1, b1, W2, b2)
    ei = edge_index0[0].astype(jnp.int32)
    ej = edge_index0[1].astype(jnp.int32)
    partials = _sc_agg(ei, ej, x, w)
    return _combine(partials, bias)


# speedup vs baseline: 9.3218x; 1.3775x over previous
import functools

import jax
import jax.numpy as jnp
from jax import lax
from jax.experimental import pallas as pl
from jax.experimental.pallas import tpu as pltpu
from jax.experimental.pallas import tpu_sc as plsc

N = 10000
E = 320000
F = 128


def _lin_body(v0_ref, wl_ref, x_ref):
    x_ref[...] = jnp.dot(v0_ref[...], wl_ref[...],
                         preferred_element_type=jnp.float32)


def _lin(v0, W_lin):
    return pl.pallas_call(
        _lin_body,
        out_shape=jax.ShapeDtypeStruct((N, F), jnp.float32),
        grid=(10,),
        in_specs=[
            pl.BlockSpec((N // 10, F), lambda i: (i, 0)),
            pl.BlockSpec((F, F), lambda i: (0, 0)),
        ],
        out_specs=pl.BlockSpec((N // 10, F), lambda i: (i, 0)),
    )(v0, W_lin)


_MLP_BM = 2000


def _mlp_body(v1_ref, w1_ref, b1_ref, w2_ref, b2_ref, w_ref):
    h = jnp.dot(v1_ref[...], w1_ref[...], preferred_element_type=jnp.float32)
    h = jnp.maximum(h + b1_ref[...], 0.0)
    w_ref[...] = jnp.dot(h, w2_ref[...],
                         preferred_element_type=jnp.float32) + b2_ref[...]


def _mlp(v1, W1, b1, W2, b2):
    ef = v1.shape[1]
    return pl.pallas_call(
        _mlp_body,
        out_shape=jax.ShapeDtypeStruct((E, F), jnp.float32),
        grid=(E // _MLP_BM,),
        in_specs=[
            pl.BlockSpec((_MLP_BM, ef), lambda i: (i, 0)),
            pl.BlockSpec((ef, F), lambda i: (0, 0)),
            pl.BlockSpec((1, F), lambda i: (0, 0)),
            pl.BlockSpec((F, F), lambda i: (0, 0)),
            pl.BlockSpec((1, F), lambda i: (0, 0)),
        ],
        out_specs=pl.BlockSpec((_MLP_BM, F), lambda i: (i, 0)),
    )(v1, W1, b1.reshape(1, F), W2, b2.reshape(1, F))


_NW = 32
_C = 40
_EPT = E // _NW
_NCHUNK = _EPT // _C
_RSTRIDE = 624
_RSIZE = 640


def _sc_agg(ei, ej, x, w):
    mesh = plsc.VectorSubcoreMesh(core_axis_name="c", subcore_axis_name="s")

    @functools.partial(
        pl.kernel,
        mesh=mesh,
        out_type=jax.ShapeDtypeStruct((2, N, F), jnp.float32),
        scratch_types=(
            [pltpu.VMEM((_C,), jnp.int32)] * 6
            + [pltpu.VMEM((_C, F), jnp.float32)] * 9
            + [pltpu.VMEM_SHARED((N, F), jnp.float32)]
            + [pltpu.SemaphoreType.DMA] * 12
        ),
    )
    def k(ei_hbm, ej_hbm, x_hbm, w_hbm, out_hbm,
          ii0, ii1, ii2, jj0, jj1, jj2,
          w0, w1, w2, xi0, xi1, xi2, xj0, xj1, xj2,
          acc_sh,
          si0, si1, si2, sw0, sw1, sw2,
          sg0, sg1, sg2, ss0, ss1, ss2):
        IB = [ii0, ii1, ii2]
        JB = [jj0, jj1, jj2]
        WB = [w0, w1, w2]
        XIB = [xi0, xi1, xi2]
        XJB = [xj0, xj1, xj2]
        SIN = [si0, si1, si2]
        SW = [sw0, sw1, sw2]
        SG = [sg0, sg1, sg2]
        SS = [ss0, ss1, ss2]

        cid = lax.axis_index("c")
        sid = lax.axis_index("s")
        wid = sid * 2 + cid

        def zbody(r, carry):
            for kk in range(8):
                xi0[r, pl.ds(kk * 16, 16)] = jnp.zeros((16,), jnp.float32)
            return carry

        lax.fori_loop(0, _C, zbody, 0)
        zbase = sid * _RSTRIDE
        for t in range(_RSIZE // _C):
            pltpu.sync_copy(xi0, acc_sh.at[pl.ds(zbase + t * _C, _C)])
        plsc.subcore_barrier()

        ebase = wid * _EPT

        def issue_idx(c, p):
            b = ebase + c * _C
            pltpu.async_copy(ei_hbm.at[pl.ds(b, _C)], IB[p], SIN[p])
            pltpu.async_copy(ej_hbm.at[pl.ds(b, _C)], JB[p], SIN[p])

        def wait_idx(p):
            pltpu.make_async_copy(ei_hbm.at[pl.ds(0, _C)], IB[p], SIN[p]).wait()
            pltpu.make_async_copy(ej_hbm.at[pl.ds(0, _C)], JB[p], SIN[p]).wait()

        def issue_w(c, p):
            b = ebase + c * _C
            pltpu.async_copy(w_hbm.at[pl.ds(b, _C)], WB[p], SW[p])

        def wait_w(p):
            pltpu.make_async_copy(w_hbm.at[pl.ds(0, _C)], WB[p], SW[p]).wait()

        def issue_gather(p):
            pltpu.async_copy(x_hbm.at[JB[p]], XJB[p], SG[p])
            pltpu.async_copy(x_hbm.at[IB[p]], XIB[p], SG[p])

        def wait_gather(p):
            pltpu.make_async_copy(x_hbm.at[JB[p]], XJB[p], SG[p]).wait()
            pltpu.make_async_copy(x_hbm.at[IB[p]], XIB[p], SG[p]).wait()

        def issue_scatter(p):
            pltpu.async_copy(XJB[p], acc_sh.at[IB[p]], SS[p], add=True)
            pltpu.async_copy(XIB[p], acc_sh.at[JB[p]], SS[p], add=True)

        def wait_scatter(p):
            pltpu.make_async_copy(XJB[p], acc_sh.at[IB[p]], SS[p]).wait()
            pltpu.make_async_copy(XIB[p], acc_sh.at[JB[p]], SS[p]).wait()

        def compute(p):
            wv, xiv, xjv = WB[p], XIB[p], XJB[p]

            def mbody(r, carry):
                for rr in range(2):
                    row = 2 * r + rr
                    for kk in range(8):
                        sl = pl.ds(kk * 16, 16)
                        ww = wv[row, sl]
                        xjv[row, sl] = xjv[row, sl] * ww
                        xiv[row, sl] = xiv[row, sl] * ww
                return carry

            lax.fori_loop(0, _C // 2, mbody, 0)

        def body(c, p, first=False, prefetch=True):
            if not first:
                wait_scatter((p + 1) % 3)
            if prefetch:
                pn = (p + 1) % 3
                issue_idx(c + 1, pn)
                issue_w(c + 1, pn)
                wait_idx(pn)
                issue_gather(pn)
            wait_gather(p)
            wait_w(p)
            compute(p)
            issue_scatter(p)

        issue_idx(0, 0)
        issue_w(0, 0)
        wait_idx(0)
        issue_gather(0)

        body(0, 0, first=True)
        body(1, 1, first=True)

        ntrip = (_NCHUNK - 2) // 3
        tail_start = 2 + 3 * ntrip

        def triple(t, carry):
            for u in range(3):
                body(2 + 3 * t + u, (2 + u) % 3)
            return carry

        lax.fori_loop(0, ntrip, triple, 0)

        for c in range(tail_start, _NCHUNK):
            body(c, c % 3, prefetch=(c < _NCHUNK - 1))
        wait_scatter((_NCHUNK - 2) % 3)
        wait_scatter((_NCHUNK - 1) % 3)

        plsc.subcore_barrier()

        obase = sid * _RSTRIDE
        pltpu.sync_copy(acc_sh.at[pl.ds(obase, _RSIZE)],
                        out_hbm.at[cid, pl.ds(obase, _RSIZE)])

    return k(ei, ej, x, w)


def _comb_body(p_ref, b_ref, o_ref):
    o_ref[...] = jnp.maximum(p_ref[0] + p_ref[1] + b_ref[...], 0.0)


def _combine(partials, bias):
    return pl.pallas_call(
        _comb_body,
        out_shape=jax.ShapeDtypeStruct((N, F), jnp.float32),
        grid=(10,),
        in_specs=[
            pl.BlockSpec((2, N // 10, F), lambda i: (0, i, 0)),
            pl.BlockSpec((1, F), lambda i: (0, 0)),
        ],
        out_specs=pl.BlockSpec((N // 10, F), lambda i: (i, 0)),
    )(partials, bias.reshape(1, F))


def kernel(v0, v1, v2, v3, edge_index0, edge_index1, edge_index2,
           W_lin, bias, W1, b1, W2, b2):
    x = _lin(v0, W_lin)
    w = _mlp(v1, W1, b1, W2, b2)
    ei = edge_index0[0].astype(jnp.int32)
    ej = edge_index0[1].astype(jnp.int32)
    partials = _sc_agg(ei, ej, x, w)
    return _combine(partials, bias)
